# baseline (speedup 1.0000x reference)
import functools

import jax
import jax.numpy as jnp
from jax import lax
from jax.experimental import pallas as pl
from jax.experimental.pallas import tpu as pltpu
from jax.experimental.pallas import tpu_sc as plsc

NUM_EMB = 103
EMB_DIM = 32
BATCH = 1024
SEQ = 50
MAX_PAD = 20

B_TOTAL = BATCH * SEQ * MAX_PAD
NUM_CORES = 2
NUM_SUBCORES = 16
NUM_WORKERS = NUM_CORES * NUM_SUBCORES
ROWS_PER_W = B_TOTAL // NUM_WORKERS
CHUNK = 1280
NCHUNK = ROWS_PER_W // CHUNK


@functools.partial(
    pl.kernel,
    out_type=jax.ShapeDtypeStruct((B_TOTAL, EMB_DIM), jnp.float32),
    mesh=plsc.VectorSubcoreMesh(core_axis_name="c", subcore_axis_name="s"),
    scratch_types=[
        pltpu.VMEM((NCHUNK, CHUNK), jnp.int32),
        pltpu.VMEM((CHUNK, EMB_DIM), jnp.float32),
        pltpu.VMEM((CHUNK, EMB_DIM), jnp.float32),
        pltpu.SemaphoreType.DMA,
        pltpu.SemaphoreType.DMA,
        pltpu.SemaphoreType.DMA,
        pltpu.SemaphoreType.DMA,
    ],
    compiler_params=pltpu.CompilerParams(use_tc_tiling_on_sc=False),
)
def _gather_rows(idx_hbm, table_hbm, out_hbm, idx_v, rows0, rows1, sg0, sg1,
                 so0, so1):
    wid = lax.axis_index("s") * NUM_CORES + lax.axis_index("c")
    base = wid * ROWS_PER_W
    rows = (rows0, rows1)
    sg = (sg0, sg1)
    so = (so0, so1)

    pltpu.sync_copy(idx_hbm.at[wid], idx_v)

    pltpu.async_copy(table_hbm.at[idx_v.at[0]], rows0, sg0)

    for g in range(NCHUNK):
        b = g % 2
        nb = (g + 1) % 2
        if g + 1 < NCHUNK:
            if g >= 1:
                pltpu.make_async_copy(rows[nb],
                                      out_hbm.at[pl.ds(base + (g - 1) * CHUNK,
                                                       CHUNK)],
                                      so[nb]).wait()
            pltpu.async_copy(table_hbm.at[idx_v.at[g + 1]], rows[nb], sg[nb])
        pltpu.make_async_copy(table_hbm.at[idx_v.at[g]], rows[b], sg[b]).wait()
        pltpu.async_copy(rows[b], out_hbm.at[pl.ds(base + g * CHUNK, CHUNK)],
                         so[b])

    last = NCHUNK - 1
    pltpu.make_async_copy(rows[(last - 1) % 2],
                          out_hbm.at[pl.ds(base + (last - 1) * CHUNK, CHUNK)],
                          so[(last - 1) % 2]).wait()
    pltpu.make_async_copy(rows[last % 2],
                          out_hbm.at[pl.ds(base + last * CHUNK, CHUNK)],
                          so[last % 2]).wait()


def kernel(indices, table):
    B, S, P = indices.shape
    idx = indices.reshape(NUM_WORKERS, NCHUNK, CHUNK).astype(jnp.int32)
    table = table.astype(jnp.float32)
    out = _gather_rows(idx, table)
    return out.reshape(B, S, P * table.shape[1])
